# baseline (speedup 1.0000x reference)
import functools

import jax
import jax.numpy as jnp
from jax import lax
from jax.experimental import pallas as pl
from jax.experimental.pallas import tpu as pltpu
from jax.experimental.pallas import tpu_sc as plsc

B = 1024
SEQ = 77
D = 512
N_PRE = 7
N_SHARE = 16
N_ATTR = 15
N_SUF = 39
SUF_START = N_PRE + N_SHARE + N_ATTR
NC = 2
NS = 16
NW = NC * NS
BPW = B // NW


def _sc_body(prefix_hbm, suffix_hbm, share_hbm, attr_hbm, vid_hbm, out_hbm,
             prefix_v, suffix_v, share_v, vid_v, attr_buf):
    wid = lax.axis_index("s") * NC + lax.axis_index("c")
    base = wid * BPW
    pltpu.sync_copy(prefix_hbm, prefix_v)
    pltpu.sync_copy(suffix_hbm, suffix_v)
    pltpu.sync_copy(share_hbm, share_v)
    pltpu.sync_copy(vid_hbm.at[pl.ds(base, BPW)], vid_v)
    for i in range(BPW):
        b = base + i
        v = vid_v[pl.ds((i // 16) * 16, 16)][i % 16]
        row0 = b * SEQ
        pltpu.sync_copy(prefix_v.at[pl.ds(v * N_PRE, N_PRE)],
                        out_hbm.at[pl.ds(row0, N_PRE)])
        pltpu.sync_copy(share_v, out_hbm.at[pl.ds(row0 + N_PRE, N_SHARE)])
        pltpu.sync_copy(attr_hbm.at[pl.ds(b * N_ATTR, N_ATTR)], attr_buf)
        pltpu.sync_copy(attr_buf,
                        out_hbm.at[pl.ds(row0 + N_PRE + N_SHARE, N_ATTR)])
        pltpu.sync_copy(suffix_v.at[pl.ds(v * N_SUF, N_SUF)],
                        out_hbm.at[pl.ds(row0 + SUF_START, N_SUF)])


@jax.jit
def _sc_call(prefix_flat, suffix_flat, share, attr_flat, vid):
    mesh = plsc.VectorSubcoreMesh(core_axis_name="c", subcore_axis_name="s")
    f = pl.kernel(
        _sc_body,
        out_type=jax.ShapeDtypeStruct((B * SEQ, D), jnp.float32),
        mesh=mesh,
        scratch_types=[
            pltpu.VMEM((3 * N_PRE, D), jnp.float32),
            pltpu.VMEM((3 * N_SUF, D), jnp.float32),
            pltpu.VMEM((N_SHARE, D), jnp.float32),
            pltpu.VMEM((BPW,), jnp.int32),
            pltpu.VMEM((N_ATTR, D), jnp.float32),
        ],
        compiler_params=pltpu.CompilerParams(use_tc_tiling_on_sc=False),
    )
    return f(prefix_flat, suffix_flat, share, attr_flat, vid)


def kernel(attribute, viewids, token_prefix, token_suffix, share_vectors):
    prefix_flat = token_prefix.reshape(3 * N_PRE, D)
    suffix_flat = token_suffix.reshape(3 * N_SUF, D)
    attr_flat = attribute.reshape(B * N_ATTR, D)
    vid = viewids.astype(jnp.int32)
    out = _sc_call(prefix_flat, suffix_flat, share_vectors, attr_flat, vid)
    return out.reshape(B, SEQ, D)


# speedup vs baseline: 1.0426x; 1.0426x over previous
import jax
import jax.numpy as jnp
from jax import lax
from jax.experimental import pallas as pl
from jax.experimental.pallas import tpu as pltpu
from jax.experimental.pallas import tpu_sc as plsc

B = 1024
SEQ = 77
D = 512
N_PRE = 7
N_SHARE = 16
N_ATTR = 15
N_SUF = 39
N_HEAD = N_PRE + N_SHARE
SUF_START = N_HEAD + N_ATTR
NC = 2
NS = 16
NW = NC * NS
BPW = B // NW
NBUF = 3


def _sc_body(prefix_hbm, suffix_hbm, share_hbm, attr_hbm, vid_hbm, out_hbm,
             head_v, suffix_v, attr_v, vid_v,
             sem_t, sem_a0, sem_a1, sem_a2, sem_b0, sem_b1, sem_b2):
    sem_a = [sem_a0, sem_a1, sem_a2]
    sem_b = [sem_b0, sem_b1, sem_b2]
    wid = lax.axis_index("s") * NC + lax.axis_index("c")
    base = wid * BPW
    pltpu.sync_copy(vid_hbm.at[pl.ds(base, BPW)], vid_v)
    pltpu.sync_copy(suffix_hbm, suffix_v)
    for v in range(3):
        pltpu.sync_copy(prefix_hbm.at[pl.ds(v * N_PRE, N_PRE)],
                        head_v.at[pl.ds(v * N_HEAD, N_PRE)])
        pltpu.sync_copy(share_hbm,
                        head_v.at[pl.ds(v * N_HEAD + N_PRE, N_SHARE)])

    def attr_in(i):
        return pltpu.async_copy(
            attr_hbm.at[pl.ds((base + i) * N_ATTR, N_ATTR)],
            attr_v.at[i % NBUF], sem_a[i % NBUF])

    h_in = [attr_in(0), attr_in(1), None]
    h_out = [None, None, None]
    h_tab = []
    for i in range(BPW):
        b = base + i
        p = i % NBUF
        v = vid_v[pl.ds((i // 16) * 16, 16)][i % 16]
        row0 = b * SEQ
        h_tab.append(pltpu.async_copy(
            head_v.at[pl.ds(v * N_HEAD, N_HEAD)],
            out_hbm.at[pl.ds(row0, N_HEAD)], sem_t))
        h_tab.append(pltpu.async_copy(
            suffix_v.at[pl.ds(v * N_SUF, N_SUF)],
            out_hbm.at[pl.ds(row0 + SUF_START, N_SUF)], sem_t))
        h_in[p].wait()
        h_out[p] = pltpu.async_copy(
            attr_v.at[p], out_hbm.at[pl.ds(row0 + N_HEAD, N_ATTR)],
            sem_b[p])
        if i + 2 < BPW:
            q = (i - 1) % NBUF
            if h_out[q] is not None:
                h_out[q].wait()
                h_out[q] = None
            h_in[q] = attr_in(i + 2)
    for h in h_out:
        if h is not None:
            h.wait()
    for h in h_tab:
        h.wait()


@jax.jit
def _sc_call(prefix_flat, suffix_flat, share, attr_flat, vid):
    mesh = plsc.VectorSubcoreMesh(core_axis_name="c", subcore_axis_name="s")
    f = pl.kernel(
        _sc_body,
        out_type=jax.ShapeDtypeStruct((B * SEQ, D), jnp.float32),
        mesh=mesh,
        scratch_types=[
            pltpu.VMEM((3 * N_HEAD, D), jnp.float32),
            pltpu.VMEM((3 * N_SUF, D), jnp.float32),
            pltpu.VMEM((NBUF, N_ATTR, D), jnp.float32),
            pltpu.VMEM((BPW,), jnp.int32),
            pltpu.SemaphoreType.DMA,
            pltpu.SemaphoreType.DMA,
            pltpu.SemaphoreType.DMA,
            pltpu.SemaphoreType.DMA,
            pltpu.SemaphoreType.DMA,
            pltpu.SemaphoreType.DMA,
            pltpu.SemaphoreType.DMA,
        ],
        compiler_params=pltpu.CompilerParams(use_tc_tiling_on_sc=False),
    )
    return f(prefix_flat, suffix_flat, share, attr_flat, vid)


def kernel(attribute, viewids, token_prefix, token_suffix, share_vectors):
    prefix_flat = token_prefix.reshape(3 * N_PRE, D)
    suffix_flat = token_suffix.reshape(3 * N_SUF, D)
    attr_flat = attribute.reshape(B * N_ATTR, D)
    vid = viewids.astype(jnp.int32)
    out = _sc_call(prefix_flat, suffix_flat, share_vectors, attr_flat, vid)
    return out.reshape(B, SEQ, D)
